# baseline (speedup 1.0000x reference)
import functools

import jax
import jax.numpy as jnp
from jax import lax
from jax.experimental import pallas as pl
from jax.experimental.pallas import tpu as pltpu
from jax.experimental.pallas import tpu_sc as plsc

EMB = 64
BATCH = 4096
SEQ = 200
NW = 32
D = 524288
COLS = 2048


def _tc_retile(table_t):
    def body(a_ref, b_ref, out_ref):
        out_ref[:, 0:64] = jnp.transpose(a_ref[...], (1, 0))
        out_ref[:, 64:128] = jnp.transpose(b_ref[...], (1, 0))

    return pl.pallas_call(
        body,
        grid=(D // COLS,),
        in_specs=[pl.BlockSpec((64, COLS), lambda i: (0, i)),
                  pl.BlockSpec((64, COLS),
                               lambda i: (0, jnp.minimum(i + D // COLS,
                                                         999999 // COLS)))],
        out_specs=pl.BlockSpec((COLS, 128), lambda i: (i, 0)),
        out_shape=jax.ShapeDtypeStruct((D, 128), jnp.float32),
    )(table_t, table_t)


def _sc_gather(tbl_rm, tok4):
    mesh = plsc.VectorSubcoreMesh(core_axis_name="c", subcore_axis_name="s")

    @functools.partial(
        pl.kernel,
        out_type=jax.ShapeDtypeStruct((SEQ, 8, NW, 8, 128), jnp.float32),
        mesh=mesh,
        scratch_types=[
            pltpu.VMEM((25, 8, 128), jnp.int32),
            [pltpu.VMEM((128, EMB), jnp.float32) for _ in range(4)],
            [pltpu.VMEM((128,), jnp.int32) for _ in range(4)],
            [pltpu.VMEM((8, 8, 128), jnp.float32) for _ in range(2)],
            [pltpu.SemaphoreType.DMA for _ in range(4)],
            [pltpu.SemaphoreType.DMA for _ in range(2)],
        ],
        compiler_params=pltpu.CompilerParams(use_tc_tiling_on_sc=False,
                                             needs_layout_passes=False),
    )
    def k(tbl_hbm, tok_hbm, out_hbm, toks, buf, qbuf, obuf, gsem, wsem):
        wid = lax.axis_index("s") * 2 + lax.axis_index("c")
        iota = lax.iota(jnp.int32, 16)
        rowv = [16 * j + iota for j in range(8)]

        pltpu.sync_copy(tok_hbm.at[:, wid], toks)

        def gather_start(kk, b):
            for j in range(8):
                t16 = toks[kk >> 3, kk & 7, pl.ds(16 * j, 16)]
                qbuf[b][pl.ds(16 * j, 16)] = (
                    ((t16 & (D - 1)) << 1) | (t16 >> 19))
            pltpu.async_copy(tbl_hbm.at[qbuf[b]], buf[b], gsem[b])

        def gather_wait(b):
            pltpu.make_async_copy(tbl_hbm.at[qbuf[b]], buf[b],
                                  gsem[b]).wait()

        def write_start(kk, ob):
            pltpu.async_copy(obuf[ob], out_hbm.at[kk, :, wid], wsem[ob])

        def write_wait(kk, ob):
            pltpu.make_async_copy(obuf[ob], out_hbm.at[kk, :, wid],
                                  wsem[ob]).wait()

        def compute(b, ob):
            def erow(e, carry):
                ev = jnp.zeros((16,), jnp.int32) + e
                for j in range(8):
                    v = plsc.load_gather(buf[b], [rowv[j], ev])
                    obuf[ob][e >> 3, e & 7, pl.ds(16 * j, 16)] = v
                return carry
            lax.fori_loop(0, EMB, erow, 0)

        for kk in range(4):
            gather_start(kk, kk)

        def body(i, carry):
            for j4 in range(4):
                kk = 4 * i + j4

                @pl.when(kk >= 2)
                def _():
                    write_wait(kk - 2, j4 % 2)

                gather_wait(j4)
                compute(j4, j4 % 2)
                write_start(kk, j4 % 2)
                gather_start(kk + 4, j4)
            return carry

        lax.fori_loop(0, 49, body, 0)
        for j4 in range(4):
            kk = 196 + j4
            write_wait(kk - 2, j4 % 2)
            gather_wait(j4)
            compute(j4, j4 % 2)
            write_start(kk, j4 % 2)
        write_wait(198, 0)
        write_wait(199, 1)

    return k(tbl_rm, tok4)


def kernel(tokens, table):
    tokens_i32 = tokens.astype(jnp.int32)
    table_t = table.T
    tok4 = (tokens_i32.reshape(NW, 128, 25, 8)
            .transpose(2, 0, 3, 1))
    tbl2 = _tc_retile(table_t)
    tbl_rm = tbl2.reshape(2 * D, EMB)
    out5 = _sc_gather(tbl_rm, tok4)
    return (out5.transpose(2, 4, 0, 1, 3)
            .reshape(BATCH, SEQ, EMB))


# speedup vs baseline: 3.2483x; 3.2483x over previous
import functools

import jax
import jax.numpy as jnp
from jax import lax
from jax.experimental import pallas as pl
from jax.experimental.pallas import tpu as pltpu
from jax.experimental.pallas import tpu_sc as plsc

EMB = 64
BATCH = 4096
SEQ = 200
NW = 32
D = 524288
COLS = 2048


def _tc_retile(table_t):
    def body(a_ref, b_ref, out_ref):
        out_ref[:, 0:64] = jnp.transpose(a_ref[...], (1, 0))
        out_ref[:, 64:128] = jnp.transpose(b_ref[...], (1, 0))

    return pl.pallas_call(
        body,
        grid=(D // COLS,),
        in_specs=[pl.BlockSpec((64, COLS), lambda i: (0, i)),
                  pl.BlockSpec((64, COLS),
                               lambda i: (0, jnp.minimum(i + D // COLS,
                                                         999999 // COLS)))],
        out_specs=pl.BlockSpec((COLS, 128), lambda i: (i, 0)),
        out_shape=jax.ShapeDtypeStruct((D, 128), jnp.float32),
    )(table_t, table_t)


def _sc_gather(tbl_rm, tok4):
    mesh = plsc.VectorSubcoreMesh(core_axis_name="c", subcore_axis_name="s")

    @functools.partial(
        pl.kernel,
        out_type=jax.ShapeDtypeStruct((SEQ, 8, NW, 8, 128), jnp.float32),
        mesh=mesh,
        scratch_types=[
            pltpu.VMEM((25, 8, 128), jnp.int32),
            [pltpu.VMEM((128, EMB), jnp.float32) for _ in range(4)],
            [pltpu.VMEM((128,), jnp.int32) for _ in range(4)],
            [pltpu.VMEM((8, 8, 128), jnp.float32) for _ in range(2)],
            [pltpu.SemaphoreType.DMA for _ in range(4)],
            [pltpu.SemaphoreType.DMA for _ in range(2)],
        ],
        compiler_params=pltpu.CompilerParams(use_tc_tiling_on_sc=False,
                                             needs_layout_passes=False),
    )
    def k(tbl_hbm, tok_hbm, out_hbm, toks, buf, qbuf, obuf, gsem, wsem):
        wid = lax.axis_index("s") * 2 + lax.axis_index("c")
        iota = lax.iota(jnp.int32, 16)
        rowv = [16 * j + iota for j in range(8)]

        pltpu.sync_copy(tok_hbm.at[:, wid], toks)

        def gather_start(kk, b):
            for j in range(8):
                t16 = toks[kk >> 3, kk & 7, pl.ds(16 * j, 16)]
                qbuf[b][pl.ds(16 * j, 16)] = (
                    ((t16 & (D - 1)) << 1) | (t16 >> 19))
            pltpu.async_copy(tbl_hbm.at[qbuf[b]], buf[b], gsem[b])

        def gather_wait(b):
            pltpu.make_async_copy(tbl_hbm.at[qbuf[b]], buf[b],
                                  gsem[b]).wait()

        def write_start(kk, ob):
            pltpu.async_copy(obuf[ob], out_hbm.at[kk, :, wid], wsem[ob])

        def write_wait(kk, ob):
            pltpu.make_async_copy(obuf[ob], out_hbm.at[kk, :, wid],
                                  wsem[ob]).wait()

        def compute(b, ob):
            @functools.partial(plsc.parallel_loop, 0, EMB, unroll=4)
            def _(e):
                ev = jnp.zeros((16,), jnp.int32) + e
                for j in range(8):
                    v = plsc.load_gather(buf[b], [rowv[j], ev])
                    obuf[ob][e >> 3, e & 7, pl.ds(16 * j, 16)] = v

        for kk in range(4):
            gather_start(kk, kk)

        def body(i, carry):
            for j4 in range(4):
                kk = 4 * i + j4

                @pl.when(kk >= 2)
                def _():
                    write_wait(kk - 2, j4 % 2)

                gather_wait(j4)
                compute(j4, j4 % 2)
                write_start(kk, j4 % 2)
                gather_start(kk + 4, j4)
            return carry

        lax.fori_loop(0, 49, body, 0)
        for j4 in range(4):
            kk = 196 + j4
            write_wait(kk - 2, j4 % 2)
            gather_wait(j4)
            compute(j4, j4 % 2)
            write_start(kk, j4 % 2)
        write_wait(198, 0)
        write_wait(199, 1)

    return k(tbl_rm, tok4)


def kernel(tokens, table):
    tokens_i32 = tokens.astype(jnp.int32)
    table_t = table.T
    tok4 = (tokens_i32.reshape(NW, 128, 25, 8)
            .transpose(2, 0, 3, 1))
    tbl2 = _tc_retile(table_t)
    tbl_rm = tbl2.reshape(2 * D, EMB)
    out5 = _sc_gather(tbl_rm, tok4)
    return (out5.transpose(2, 4, 0, 1, 3)
            .reshape(BATCH, SEQ, EMB))


# speedup vs baseline: 4.8717x; 1.4998x over previous
import functools

import jax
import jax.numpy as jnp
from jax import lax
from jax.experimental import pallas as pl
from jax.experimental.pallas import tpu as pltpu
from jax.experimental.pallas import tpu_sc as plsc

EMB = 64
BATCH = 4096
SEQ = 200
NW = 32
D = 524288
COLS = 8192


def _tc_retile(table_t):
    def body(a_ref, b_ref, out_ref):
        x = jnp.concatenate([a_ref[...], b_ref[...]], axis=0)
        out_ref[...] = jnp.transpose(x, (1, 0))

    return pl.pallas_call(
        body,
        grid=(D // COLS,),
        in_specs=[pl.BlockSpec((64, COLS), lambda i: (0, i)),
                  pl.BlockSpec((64, COLS),
                               lambda i: (0, jnp.minimum(i + D // COLS,
                                                         999999 // COLS)))],
        out_specs=pl.BlockSpec((COLS, 128), lambda i: (i, 0)),
        out_shape=jax.ShapeDtypeStruct((D, 128), jnp.float32),
    )(table_t, table_t)


def _sc_gather(tbl_rm, tok4):
    mesh = plsc.VectorSubcoreMesh(core_axis_name="c", subcore_axis_name="s")

    @functools.partial(
        pl.kernel,
        out_type=jax.ShapeDtypeStruct((SEQ, 8, NW, 8, 128), jnp.float32),
        mesh=mesh,
        scratch_types=[
            pltpu.VMEM((25, 8, 128), jnp.int32),
            [pltpu.VMEM((128, EMB), jnp.float32) for _ in range(4)],
            [pltpu.VMEM((128,), jnp.int32) for _ in range(4)],
            [pltpu.VMEM((8, 8, 128), jnp.float32) for _ in range(2)],
            [pltpu.SemaphoreType.DMA for _ in range(4)],
            [pltpu.SemaphoreType.DMA for _ in range(2)],
        ],
        compiler_params=pltpu.CompilerParams(use_tc_tiling_on_sc=False,
                                             needs_layout_passes=False),
    )
    def k(tbl_hbm, tok_hbm, out_hbm, toks, buf, qbuf, obuf, gsem, wsem):
        wid = lax.axis_index("s") * 2 + lax.axis_index("c")
        iota = lax.iota(jnp.int32, 16)
        rowv = [16 * j + iota for j in range(8)]

        pltpu.sync_copy(tok_hbm.at[:, wid], toks)

        def gather_start(kk, b):
            for j in range(8):
                t16 = toks[kk >> 3, kk & 7, pl.ds(16 * j, 16)]
                qbuf[b][pl.ds(16 * j, 16)] = (
                    ((t16 & (D - 1)) << 1) | (t16 >> 19))
            pltpu.async_copy(tbl_hbm.at[qbuf[b]], buf[b], gsem[b])

        def gather_wait(b):
            pltpu.make_async_copy(tbl_hbm.at[qbuf[b]], buf[b],
                                  gsem[b]).wait()

        def write_start(kk, ob):
            pltpu.async_copy(obuf[ob], out_hbm.at[kk, :, wid], wsem[ob])

        def write_wait(kk, ob):
            pltpu.make_async_copy(obuf[ob], out_hbm.at[kk, :, wid],
                                  wsem[ob]).wait()

        def compute(b, ob):
            @functools.partial(plsc.parallel_loop, 0, EMB, unroll=4)
            def _(e):
                ev = jnp.zeros((16,), jnp.int32) + e
                for j in range(8):
                    v = plsc.load_gather(buf[b], [rowv[j], ev])
                    obuf[ob][e >> 3, e & 7, pl.ds(16 * j, 16)] = v

        for kk in range(4):
            gather_start(kk, kk)

        def body(i, carry):
            for j4 in range(4):
                kk = 4 * i + j4

                @pl.when(kk >= 2)
                def _():
                    write_wait(kk - 2, j4 % 2)

                gather_wait(j4)
                compute(j4, j4 % 2)
                write_start(kk, j4 % 2)
                gather_start(kk + 4, j4)
            return carry

        lax.fori_loop(0, 49, body, 0)
        for j4 in range(4):
            kk = 196 + j4
            write_wait(kk - 2, j4 % 2)
            gather_wait(j4)
            compute(j4, j4 % 2)
            write_start(kk, j4 % 2)
        write_wait(198, 0)
        write_wait(199, 1)

    return k(tbl_rm, tok4)


def kernel(tokens, table):
    tokens_i32 = tokens.astype(jnp.int32)
    table_t = table.T
    tok4 = (tokens_i32.reshape(NW, 128, 25, 8)
            .transpose(2, 0, 3, 1))
    tbl2 = _tc_retile(table_t)
    tbl_rm = tbl2.reshape(2 * D, EMB)
    out5 = _sc_gather(tbl_rm, tok4)
    return (out5.transpose(2, 4, 0, 1, 3)
            .reshape(BATCH, SEQ, EMB))
